# baseline (speedup 1.0000x reference)
import jax
import jax.numpy as jnp
from jax.experimental import pallas as pl
from jax.experimental.pallas import tpu as pltpu
from jax.experimental.pallas import tpu_sc as plsc

B = 1024
KNN = 16
N = 100000
NBLK = 2048
NBLOCKS = 49
NPAD = NBLK * NBLOCKS

BIG = 1e30
IBIG = 2**31 - 1


NCLS = 128
NSUB = NBLK // NCLS


def _topk_body(x_ref, mkT_ref, oi_ref, bd_ref, bi_ref,
               gm_ref, gm2_ref, gi_ref, gi2_ref):
    j = pl.program_id(0)

    @pl.when(j == 0)
    def _():
        bd_ref[...] = jnp.full((B, KNN), BIG, jnp.float32)
        bi_ref[...] = jnp.full((B, KNN), IBIG, jnp.int32)

    xb = x_ref[...]
    kb = mkT_ref[...]
    x0 = xb[:, 0:1]
    x1 = xb[:, 1:2]
    xx = x0 * x0 + x1 * x1
    x0b = x0.astype(jnp.bfloat16).astype(jnp.float32)
    x1b = x1.astype(jnp.bfloat16).astype(jnp.float32)

    def d2_slice(s):
        k0 = kb[0:1, s * NCLS:(s + 1) * NCLS]
        k1 = kb[1:2, s * NCLS:(s + 1) * NCLS]
        kk = k0 * k0 + k1 * k1
        k0b = k0.astype(jnp.bfloat16).astype(jnp.float32)
        k1b = k1.astype(jnp.bfloat16).astype(jnp.float32)
        xy = x0b * k0b + x1b * k1b
        d2s = (xx + kk) - 2.0 * xy
        ci = iota128 + (j * NBLK + s * NCLS)
        return jnp.where(ci >= N, BIG, d2s), ci

    iota128 = jax.lax.broadcasted_iota(jnp.int32, (B, NCLS), 1)
    w = bd_ref[:, KNN - 1:KNN]

    gm, gi = d2_slice(0)
    cnt = (gm < w).astype(jnp.int32)
    gm2 = jnp.full((B, NCLS), BIG, jnp.float32)
    gi2 = jnp.full((B, NCLS), IBIG, jnp.int32)
    for s in range(1, NSUB):
        c, ci = d2_slice(s)
        cnt = cnt + (c < w).astype(jnp.int32)
        lt = c < gm
        lose_v = jnp.where(lt, gm, c)
        lose_i = jnp.where(lt, gi, ci)
        gm = jnp.where(lt, c, gm)
        gi = jnp.where(lt, ci, gi)
        lt2 = jnp.logical_or(
            lose_v < gm2,
            jnp.logical_and(lose_v == gm2, lose_i < gi2))
        gm2 = jnp.where(lt2, lose_v, gm2)
        gi2 = jnp.where(lt2, lose_i, gi2)
    hq = jnp.sum(cnt, axis=1, keepdims=True)
    gmax = jnp.max(cnt)
    hmax = jnp.max(hq)
    bad = jnp.logical_or(gmax > 2, hmax > KNN)

    @pl.when(bad)
    def _():
        parts_d = [bd_ref[...], jnp.full((B, 128 - KNN), BIG, jnp.float32)]
        parts_i = [bi_ref[...], jnp.full((B, 128 - KNN), IBIG, jnp.int32)]
        for s in range(NSUB):
            ds, cs = d2_slice(s)
            parts_d.append(ds)
            parts_i.append(cs)
        catd = jnp.concatenate(parts_d, axis=1)
        cati = jnp.concatenate(parts_i, axis=1)
        newd = []
        newi = []
        for _t in range(KNN):
            m = jnp.min(catd, axis=1, keepdims=True)
            pick = jnp.min(jnp.where(catd == m, cati, IBIG), axis=1,
                           keepdims=True)
            newd.append(m)
            newi.append(pick)
            catd = jnp.where(cati == pick, BIG, catd)
        bd_ref[...] = jnp.concatenate(newd, axis=1)
        bi_ref[...] = jnp.concatenate(newi, axis=1)

    def extract8(gm, gi, gm2, gi2, bd, bi):
        lane16 = jax.lax.broadcasted_iota(jnp.int32, (B, KNN), 1)
        for _ in range(8):
            m = jnp.min(gm, axis=1, keepdims=True)
            pick = jnp.min(jnp.where(gm == m, gi, IBIG), axis=1,
                           keepdims=True)
            sel = gi == pick
            gm = jnp.where(sel, gm2, gm)
            gi = jnp.where(sel, gi2, gi)
            gm2 = jnp.where(sel, BIG, gm2)
            gi2 = jnp.where(sel, IBIG, gi2)
            pos = jnp.sum((bd <= m).astype(jnp.int32), axis=1, keepdims=True)
            bsh = jnp.concatenate([bd[:, :1], bd[:, :KNN - 1]], axis=1)
            ish = jnp.concatenate([bi[:, :1], bi[:, :KNN - 1]], axis=1)
            bd = jnp.where(lane16 < pos, bd,
                           jnp.where(lane16 == pos, m, bsh))
            bi = jnp.where(lane16 < pos, bi,
                           jnp.where(lane16 == pos, pick, ish))
        return gm, gi, gm2, gi2, bd, bi

    @pl.when(jnp.logical_not(bad))
    def _():
        ngm, ngi, ngm2, ngi2, bd, bi = extract8(
            gm, gi, gm2, gi2, bd_ref[...], bi_ref[...])
        gm_ref[...] = ngm
        gi_ref[...] = ngi
        gm2_ref[...] = ngm2
        gi2_ref[...] = ngi2
        bd_ref[...] = bd
        bi_ref[...] = bi

    @pl.when(jnp.logical_and(jnp.logical_not(bad), hmax > 8))
    def _():
        _, _, _, _, bd, bi = extract8(
            gm_ref[...], gi_ref[...], gm2_ref[...], gi2_ref[...],
            bd_ref[...], bi_ref[...])
        bd_ref[...] = bd
        bi_ref[...] = bi

    @pl.when(j == NBLOCKS - 1)
    def _():
        oi_ref[...] = bi_ref[...]


def _topk_indices(x, mkT_padded):
    return pl.pallas_call(
        _topk_body,
        grid=(NBLOCKS,),
        in_specs=[
            pl.BlockSpec((B, 2), lambda j: (0, 0)),
            pl.BlockSpec((2, NBLK), lambda j: (0, j)),
        ],
        out_specs=pl.BlockSpec((B, KNN), lambda j: (0, 0)),
        out_shape=jax.ShapeDtypeStruct((B, KNN), jnp.int32),
        scratch_shapes=[
            pltpu.VMEM((B, KNN), jnp.float32),
            pltpu.VMEM((B, KNN), jnp.int32),
            pltpu.VMEM((B, NCLS), jnp.float32),
            pltpu.VMEM((B, NCLS), jnp.float32),
            pltpu.VMEM((B, NCLS), jnp.int32),
            pltpu.VMEM((B, NCLS), jnp.int32),
        ],
        compiler_params=pltpu.CompilerParams(
            dimension_semantics=("arbitrary",),
        ),
    )(x, mkT_padded)


def _sc_gather(table, idx):
    n_idx = idx.shape[0]
    window = 128
    idx2 = idx.reshape(1, n_idx)
    mesh = plsc.VectorSubcoreMesh(core_axis_name="core",
                                  subcore_axis_name="subcore")

    @pl.kernel(out_type=jax.ShapeDtypeStruct((n_idx, 128), jnp.float32),
               mesh=mesh)
    def kern(tab_hbm, i_hbm, o_hbm):
        def body(i_vmem, o_vmem):
            pltpu.sync_copy(tab_hbm.at[i_vmem.at[0]], o_vmem)

        pltpu.emit_pipeline(
            body,
            grid=(n_idx // window,),
            in_specs=[pl.BlockSpec((1, window), lambda i: (0, i))],
            out_specs=[pl.BlockSpec((window, 128), lambda i: (i, 0))],
            core_axis_name=("core", "subcore"),
            dimension_semantics=(pltpu.PARALLEL,),
        )(i_hbm, o_hbm)

    return kern(table, idx2)


def _bn_relu(h, g, beta):
    m = jnp.mean(h, axis=0, keepdims=True)
    c = h - m
    v = jnp.mean(c * c, axis=0, keepdims=True)
    return jax.nn.relu(c / jnp.sqrt(v + 1e-5) * g + beta)


def _mlp_body(kf_ref,
              w1_ref, b1_ref, w2_ref, b2_ref, w3_ref, b3_ref,
              w4_ref, b4_ref, w5_ref, b5_ref, w6_ref, b6_ref,
              g1_ref, be1_ref, g2_ref, be2_ref, g3_ref, be3_ref,
              g4_ref, be4_ref, g5_ref, be5_ref, o_ref):
    h = kf_ref[...]
    ws = [w1_ref, w2_ref, w3_ref, w4_ref, w5_ref]
    bs = [b1_ref, b2_ref, b3_ref, b4_ref, b5_ref]
    gs = [g1_ref, g2_ref, g3_ref, g4_ref, g5_ref]
    bes = [be1_ref, be2_ref, be3_ref, be4_ref, be5_ref]
    for w, b, g, be in zip(ws, bs, gs, bes):
        h = jnp.dot(h.astype(jnp.bfloat16), w[...].astype(jnp.bfloat16),
                    preferred_element_type=jnp.float32) + b[...]
        h = _bn_relu(h, g[...], be[...])
    o_ref[...] = (jnp.dot(h.astype(jnp.bfloat16),
                          w6_ref[...].astype(jnp.bfloat16),
                          preferred_element_type=jnp.float32) + b6_ref[...])


def _mlp(kf, args2d):
    return pl.pallas_call(
        _mlp_body,
        out_shape=jax.ShapeDtypeStruct((B, 2), jnp.float32),
    )(kf, *args2d)


def kernel(x, measure_keys, measure_vals, W1, b1, W2, b2, W3, b3, W4, b4,
           W5, b5, W6, b6, g1, beta1, g2, beta2, g3, beta3, g4, beta4,
           g5, beta5):
    mkT = jnp.pad(measure_keys.T, ((0, 0), (0, NPAD - N)))
    idcs = _topk_indices(x, mkT)

    table = jnp.pad(
        jnp.concatenate([measure_keys, measure_vals], axis=1),
        ((0, 0), (0, 124)))
    nb = _sc_gather(table, idcs.reshape(-1))[:, :4]
    knn_feat = jnp.concatenate([x, nb.reshape(B, 4 * KNN)], axis=1)

    args2d = [W1, b1.reshape(1, -1), W2, b2.reshape(1, -1),
              W3, b3.reshape(1, -1), W4, b4.reshape(1, -1),
              W5, b5.reshape(1, -1), W6, b6.reshape(1, -1),
              g1.reshape(1, -1), beta1.reshape(1, -1),
              g2.reshape(1, -1), beta2.reshape(1, -1),
              g3.reshape(1, -1), beta3.reshape(1, -1),
              g4.reshape(1, -1), beta4.reshape(1, -1),
              g5.reshape(1, -1), beta5.reshape(1, -1)]
    out = _mlp(knn_feat, args2d)
    return out, knn_feat


# speedup vs baseline: 1.6605x; 1.6605x over previous
import jax
import jax.numpy as jnp
from jax.experimental import pallas as pl
from jax.experimental.pallas import tpu as pltpu
from jax.experimental.pallas import tpu_sc as plsc

B = 1024
KNN = 16
N = 100000
NBLK = 2048
NBLOCKS = 49
NPAD = NBLK * NBLOCKS

BIG = 1e30
IBIG = 2**31 - 1


NCLS = 128
NSUB = NBLK // NCLS


def _topk_body(x_ref, mkT_ref, oi_ref, bd_ref, bi_ref,
               gm_ref, gm2_ref, gi_ref, gi2_ref):
    j = pl.program_id(0)

    @pl.when(j == 0)
    def _():
        bd_ref[...] = jnp.full((B, KNN), BIG, jnp.float32)
        bi_ref[...] = jnp.full((B, KNN), IBIG, jnp.int32)

    xb = x_ref[...]
    kb = mkT_ref[...]
    x0 = xb[:, 0:1]
    x1 = xb[:, 1:2]
    xx = x0 * x0 + x1 * x1
    x0b = x0.astype(jnp.bfloat16).astype(jnp.float32)
    x1b = x1.astype(jnp.bfloat16).astype(jnp.float32)

    def d2_slice(s):
        k0 = kb[0:1, s * NCLS:(s + 1) * NCLS]
        k1 = kb[1:2, s * NCLS:(s + 1) * NCLS]
        kk = k0 * k0 + k1 * k1
        k0b = k0.astype(jnp.bfloat16).astype(jnp.float32)
        k1b = k1.astype(jnp.bfloat16).astype(jnp.float32)
        xy = x0b * k0b + x1b * k1b
        d2s = (xx + kk) - 2.0 * xy
        ci = iota128 + (j * NBLK + s * NCLS)
        return jnp.where(ci >= N, BIG, d2s), ci

    iota128 = jax.lax.broadcasted_iota(jnp.int32, (B, NCLS), 1)
    w = bd_ref[:, KNN - 1:KNN]

    gm, gi = d2_slice(0)
    cnt = (gm < w).astype(jnp.int32)
    gm2 = jnp.full((B, NCLS), BIG, jnp.float32)
    gi2 = jnp.full((B, NCLS), IBIG, jnp.int32)
    for s in range(1, NSUB):
        c, ci = d2_slice(s)
        cnt = cnt + (c < w).astype(jnp.int32)
        lt = c < gm
        lose_v = jnp.where(lt, gm, c)
        lose_i = jnp.where(lt, gi, ci)
        gm = jnp.where(lt, c, gm)
        gi = jnp.where(lt, ci, gi)
        lt2 = jnp.logical_or(
            lose_v < gm2,
            jnp.logical_and(lose_v == gm2, lose_i < gi2))
        gm2 = jnp.where(lt2, lose_v, gm2)
        gi2 = jnp.where(lt2, lose_i, gi2)
    hq = jnp.sum(cnt, axis=1, keepdims=True)
    gmax = jnp.max(cnt)
    hmax = jnp.max(hq)
    HCAP = 10
    bad = jnp.logical_or(gmax > 2, hmax > HCAP)

    @pl.when(bad)
    def _():
        parts_d = [bd_ref[...], jnp.full((B, 128 - KNN), BIG, jnp.float32)]
        parts_i = [bi_ref[...], jnp.full((B, 128 - KNN), IBIG, jnp.int32)]
        for s in range(NSUB):
            ds, cs = d2_slice(s)
            parts_d.append(ds)
            parts_i.append(cs)
        catd = jnp.concatenate(parts_d, axis=1)
        cati = jnp.concatenate(parts_i, axis=1)
        newd = []
        newi = []
        for _t in range(KNN):
            m = jnp.min(catd, axis=1, keepdims=True)
            pick = jnp.min(jnp.where(catd == m, cati, IBIG), axis=1,
                           keepdims=True)
            newd.append(m)
            newi.append(pick)
            catd = jnp.where(cati == pick, BIG, catd)
        bd_ref[...] = jnp.concatenate(newd, axis=1)
        bi_ref[...] = jnp.concatenate(newi, axis=1)

    @pl.when(jnp.logical_not(bad))
    def _():
        gm_ref[...] = gm
        gm2_ref[...] = gm2
        gi_ref[...] = gi
        gi2_ref[...] = gi2

    lane16 = jax.lax.broadcasted_iota(jnp.int32, (B, KNN), 1)
    for g in range(HCAP // 2):
        @pl.when(jnp.logical_and(jnp.logical_not(bad), hmax > 2 * g))
        def _():
            gm = gm_ref[...]
            gi = gi_ref[...]
            gm2 = gm2_ref[...]
            gi2 = gi2_ref[...]
            bd = bd_ref[...]
            bi = bi_ref[...]
            for _ in range(2):
                m = jnp.min(gm, axis=1, keepdims=True)
                pick = jnp.min(jnp.where(gm == m, gi, IBIG), axis=1,
                               keepdims=True)
                sel = gi == pick
                gm = jnp.where(sel, gm2, gm)
                gi = jnp.where(sel, gi2, gi)
                gm2 = jnp.where(sel, BIG, gm2)
                gi2 = jnp.where(sel, IBIG, gi2)
                pos = jnp.sum((bd <= m).astype(jnp.int32), axis=1,
                              keepdims=True)
                bsh = jnp.concatenate([bd[:, :1], bd[:, :KNN - 1]], axis=1)
                ish = jnp.concatenate([bi[:, :1], bi[:, :KNN - 1]], axis=1)
                bd = jnp.where(lane16 < pos, bd,
                               jnp.where(lane16 == pos, m, bsh))
                bi = jnp.where(lane16 < pos, bi,
                               jnp.where(lane16 == pos, pick, ish))
            gm_ref[...] = gm
            gi_ref[...] = gi
            gm2_ref[...] = gm2
            gi2_ref[...] = gi2
            bd_ref[...] = bd
            bi_ref[...] = bi

    @pl.when(j == NBLOCKS - 1)
    def _():
        oi_ref[...] = bi_ref[...]


def _topk_indices(x, mkT_padded):
    return pl.pallas_call(
        _topk_body,
        grid=(NBLOCKS,),
        in_specs=[
            pl.BlockSpec((B, 2), lambda j: (0, 0)),
            pl.BlockSpec((2, NBLK), lambda j: (0, j)),
        ],
        out_specs=pl.BlockSpec((B, KNN), lambda j: (0, 0)),
        out_shape=jax.ShapeDtypeStruct((B, KNN), jnp.int32),
        scratch_shapes=[
            pltpu.VMEM((B, KNN), jnp.float32),
            pltpu.VMEM((B, KNN), jnp.int32),
            pltpu.VMEM((B, NCLS), jnp.float32),
            pltpu.VMEM((B, NCLS), jnp.float32),
            pltpu.VMEM((B, NCLS), jnp.int32),
            pltpu.VMEM((B, NCLS), jnp.int32),
        ],
        compiler_params=pltpu.CompilerParams(
            dimension_semantics=("arbitrary",),
        ),
    )(x, mkT_padded)


def _sc_gather(table, idx):
    n_idx = idx.shape[0]
    window = 128
    idx2 = idx.reshape(1, n_idx)
    mesh = plsc.VectorSubcoreMesh(core_axis_name="core",
                                  subcore_axis_name="subcore")

    @pl.kernel(out_type=jax.ShapeDtypeStruct((n_idx, 128), jnp.float32),
               mesh=mesh)
    def kern(tab_hbm, i_hbm, o_hbm):
        def body(i_vmem, o_vmem):
            pltpu.sync_copy(tab_hbm.at[i_vmem.at[0]], o_vmem)

        pltpu.emit_pipeline(
            body,
            grid=(n_idx // window,),
            in_specs=[pl.BlockSpec((1, window), lambda i: (0, i))],
            out_specs=[pl.BlockSpec((window, 128), lambda i: (i, 0))],
            core_axis_name=("core", "subcore"),
            dimension_semantics=(pltpu.PARALLEL,),
        )(i_hbm, o_hbm)

    return kern(table, idx2)


def _bn_relu(h, g, beta):
    m = jnp.mean(h, axis=0, keepdims=True)
    c = h - m
    v = jnp.mean(c * c, axis=0, keepdims=True)
    return jax.nn.relu(c / jnp.sqrt(v + 1e-5) * g + beta)


def _mlp_body(kf_ref,
              w1_ref, b1_ref, w2_ref, b2_ref, w3_ref, b3_ref,
              w4_ref, b4_ref, w5_ref, b5_ref, w6_ref, b6_ref,
              g1_ref, be1_ref, g2_ref, be2_ref, g3_ref, be3_ref,
              g4_ref, be4_ref, g5_ref, be5_ref, o_ref):
    h = kf_ref[...]
    ws = [w1_ref, w2_ref, w3_ref, w4_ref, w5_ref]
    bs = [b1_ref, b2_ref, b3_ref, b4_ref, b5_ref]
    gs = [g1_ref, g2_ref, g3_ref, g4_ref, g5_ref]
    bes = [be1_ref, be2_ref, be3_ref, be4_ref, be5_ref]
    for w, b, g, be in zip(ws, bs, gs, bes):
        h = jnp.dot(h.astype(jnp.bfloat16), w[...].astype(jnp.bfloat16),
                    preferred_element_type=jnp.float32) + b[...]
        h = _bn_relu(h, g[...], be[...])
    o_ref[...] = (jnp.dot(h.astype(jnp.bfloat16),
                          w6_ref[...].astype(jnp.bfloat16),
                          preferred_element_type=jnp.float32) + b6_ref[...])


def _mlp(kf, args2d):
    return pl.pallas_call(
        _mlp_body,
        out_shape=jax.ShapeDtypeStruct((B, 2), jnp.float32),
    )(kf, *args2d)


def kernel(x, measure_keys, measure_vals, W1, b1, W2, b2, W3, b3, W4, b4,
           W5, b5, W6, b6, g1, beta1, g2, beta2, g3, beta3, g4, beta4,
           g5, beta5):
    mkT = jnp.pad(measure_keys.T, ((0, 0), (0, NPAD - N)))
    idcs = _topk_indices(x, mkT)

    table = jnp.pad(
        jnp.concatenate([measure_keys, measure_vals], axis=1),
        ((0, 0), (0, 124)))
    nb = _sc_gather(table, idcs.reshape(-1))[:, :4]
    knn_feat = jnp.concatenate([x, nb.reshape(B, 4 * KNN)], axis=1)

    args2d = [W1, b1.reshape(1, -1), W2, b2.reshape(1, -1),
              W3, b3.reshape(1, -1), W4, b4.reshape(1, -1),
              W5, b5.reshape(1, -1), W6, b6.reshape(1, -1),
              g1.reshape(1, -1), beta1.reshape(1, -1),
              g2.reshape(1, -1), beta2.reshape(1, -1),
              g3.reshape(1, -1), beta3.reshape(1, -1),
              g4.reshape(1, -1), beta4.reshape(1, -1),
              g5.reshape(1, -1), beta5.reshape(1, -1)]
    out = _mlp(knn_feat, args2d)
    return out, knn_feat
